# baseline (speedup 1.0000x reference)
import functools

import jax
import jax.numpy as jnp
import numpy as np
from jax import lax
from jax.experimental import pallas as pl
from jax.experimental.pallas import tpu as pltpu
from jax.experimental.pallas import tpu_sc as plsc

_FEAT_DIM = 2
_TD = 4
_NL = 4
_RES = [16 * 2**l + 1 for l in range(_NL)]
_SIZES = [r * r * r for r in _RES]
_OFFS = [0]
for _s in _SIZES:
    _OFFS.append(_OFFS[-1] + _s)
_B = 10000
_NW = 32
_BT = 320
_BP = _NW * _BT
_NG = _BT // 16

_TS = np.arange(_TD, dtype=np.float32) * np.float32(1.0 / (_TD - 1))
_T1000 = np.array([_TS[1], _TS[0], _TS[0], _TS[0]], np.float32)
_T2211 = np.array([_TS[2], _TS[2], _TS[1], _TS[1]], np.float32)
_T3332 = np.array([_TS[3], _TS[3], _TS[3], _TS[2]], np.float32)
_DENOM = (_TS - _T1000) * (_TS - _T2211) * (_TS - _T3332)


def _sc_encode_fn():
    mesh = plsc.VectorSubcoreMesh(
        core_axis_name="c", subcore_axis_name="s", num_cores=2, num_subcores=16
    )

    @functools.partial(
        pl.kernel,
        out_type=jax.ShapeDtypeStruct((8 * _BP,), jnp.float32),
        mesh=mesh,
        scratch_types=[
            pltpu.VMEM((_BT,), jnp.float32),
            pltpu.VMEM((_BT,), jnp.float32),
            pltpu.VMEM((_BT,), jnp.float32),
            pltpu.VMEM((_BT,), jnp.float32),
            pltpu.VMEM((_NG * 4, 128), jnp.int32),
            pltpu.VMEM((_NG * 4, 128, 8), jnp.float32),
            pltpu.VMEM((8, _BT), jnp.float32),
            pltpu.SemaphoreType.DMA,
            pltpu.SemaphoreType.DMA,
        ],
        compiler_params=pltpu.CompilerParams(
            needs_layout_passes=False, use_tc_tiling_on_sc=False
        ),
    )
    def enc(xs_h, ys_h, zs_h, ts_h, feats_h, out_h,
            xs_v, ys_v, zs_v, ts_v, idx_v, rows_v, out_v, sem0, sem1):
        wid = lax.axis_index("s") * 2 + lax.axis_index("c")
        base = wid * _BT
        pltpu.sync_copy(xs_h.at[pl.ds(base, _BT)], xs_v)
        pltpu.sync_copy(ys_h.at[pl.ds(base, _BT)], ys_v)
        pltpu.sync_copy(zs_h.at[pl.ds(base, _BT)], zs_v)
        pltpu.sync_copy(ts_h.at[pl.ds(base, _BT)], ts_v)

        iota = lax.iota(jnp.int32, 16)

        def load16(ref, b0):
            return ref[pl.ds(b0, 16)]

        def fracs(g):
            b0 = g * 16
            x = load16(xs_v, b0)
            y = load16(ys_v, b0)
            z = load16(zs_v, b0)
            out = []
            for l in range(_NL):
                r = _RES[l]
                inv = float(r - 1)
                px = x * inv
                py = y * inv
                pz = z * inv
                bx = jnp.clip(px.astype(jnp.int32), 0, r - 2)
                by = jnp.clip(py.astype(jnp.int32), 0, r - 2)
                bz = jnp.clip(pz.astype(jnp.int32), 0, r - 2)
                fx = px - bx.astype(jnp.float32)
                fy = py - by.astype(jnp.float32)
                fz = pz - bz.astype(jnp.float32)
                out.append((bx, by, bz, fx, fy, fz))
            return out

        def compute_indices(g):
            fr = fracs(g)
            for l in range(_NL):
                r = _RES[l]
                bx, by, bz, _, _, _ = fr[l]
                lin = bx * (r * r) + by * r + bz + _OFFS[l]
                row = g * 4 + l
                for c in range(8):
                    u, v, w = c >> 2, (c >> 1) & 1, c & 1
                    idx_v[row, pl.ds(c * 16, 16)] = lin + (u * (r * r) + v * r + w)

        def fire(g, sem):
            for l in range(_NL):
                row = g * 4 + l
                pltpu.async_copy(feats_h.at[idx_v.at[row]], rows_v.at[row], sem)

        def drain(g, sem):
            for l in range(_NL):
                row = g * 4 + l
                pltpu.make_async_copy(
                    feats_h.at[idx_v.at[row]], rows_v.at[row], sem
                ).wait()

        def compute_out(g):
            b0 = g * 16
            t = load16(ts_v, b0)
            d = [t - float(ts_k) for ts_k in _TS]
            vb = [
                (d[1] * d[2] * d[3]) / float(_DENOM[0]),
                (d[0] * d[2] * d[3]) / float(_DENOM[1]),
                (d[0] * d[1] * d[3]) / float(_DENOM[2]),
                (d[0] * d[1] * d[2]) / float(_DENOM[3]),
            ]
            fr = fracs(g)
            for l in range(_NL):
                _, _, _, fx, fy, fz = fr[l]
                wx = [1.0 - fx, fx]
                wy = [1.0 - fy, fy]
                wz = [1.0 - fz, fz]
                row = g * 4 + l
                ridx = jnp.full((16,), 0, jnp.int32) + row
                acc0 = jnp.zeros((16,), jnp.float32)
                acc1 = jnp.zeros((16,), jnp.float32)
                for c in range(8):
                    u, v, w = c >> 2, (c >> 1) & 1, c & 1
                    W = wx[u] * wy[v] * wz[w]
                    cidx = iota + (c * 16)
                    e = [
                        plsc.load_gather(
                            rows_v,
                            [ridx, cidx, jnp.full((16,), k * 2 + j, jnp.int32)],
                        )
                        for k in range(_TD)
                        for j in range(2)
                    ]
                    ft0 = vb[0] * e[0] + vb[1] * e[2] + vb[2] * e[4] + vb[3] * e[6]
                    ft1 = vb[0] * e[1] + vb[1] * e[3] + vb[2] * e[5] + vb[3] * e[7]
                    acc0 = acc0 + W * ft0
                    acc1 = acc1 + W * ft1
                out_v[2 * l, pl.ds(b0, 16)] = acc0
                out_v[2 * l + 1, pl.ds(b0, 16)] = acc1

        compute_indices(0)
        fire(0, sem0)
        compute_indices(1)
        fire(1, sem1)

        def body(i, carry):
            g = 2 * i
            drain(g, sem0)
            compute_indices(g + 2)
            fire(g + 2, sem0)
            compute_out(g)
            drain(g + 1, sem1)
            compute_indices(g + 3)
            fire(g + 3, sem1)
            compute_out(g + 1)
            return carry

        lax.fori_loop(0, (_NG - 2) // 2, body, 0)
        drain(_NG - 2, sem0)
        compute_out(_NG - 2)
        drain(_NG - 1, sem1)
        compute_out(_NG - 1)

        for rrow in range(8):
            pltpu.sync_copy(out_v.at[rrow], out_h.at[pl.ds(rrow * _BP + base, _BT)])

    return enc


_ENC_CACHE = []


@jax.jit
def kernel(positions, feats):
    if not _ENC_CACHE:
        _ENC_CACHE.append(_sc_encode_fn())
    pos = jnp.pad(positions, ((0, _BP - _B), (0, 0)))
    pos_t = pos.T
    out_flat = _ENC_CACHE[0](pos_t[0], pos_t[1], pos_t[2], pos_t[3], feats)
    out = out_flat.reshape(8, _BP).T
    return out[:_B]


# speedup vs baseline: 2.1491x; 2.1491x over previous
import functools

import jax
import jax.numpy as jnp
import numpy as np
from jax import lax
from jax.experimental import pallas as pl
from jax.experimental.pallas import tpu as pltpu
from jax.experimental.pallas import tpu_sc as plsc

_FEAT_DIM = 2
_TD = 4
_NL = 4
_RES = [16 * 2**l + 1 for l in range(_NL)]
_SIZES = [r * r * r for r in _RES]
_OFFS = [0]
for _s in _SIZES:
    _OFFS.append(_OFFS[-1] + _s)
_B = 10000
_NW = 32
_BT = 320
_BP = _NW * _BT
_NG = _BT // 16

_TS = np.arange(_TD, dtype=np.float32) * np.float32(1.0 / (_TD - 1))
_T1000 = np.array([_TS[1], _TS[0], _TS[0], _TS[0]], np.float32)
_T2211 = np.array([_TS[2], _TS[2], _TS[1], _TS[1]], np.float32)
_T3332 = np.array([_TS[3], _TS[3], _TS[3], _TS[2]], np.float32)
_DENOM = (_TS - _T1000) * (_TS - _T2211) * (_TS - _T3332)


def _sc_encode_fn():
    mesh = plsc.VectorSubcoreMesh(
        core_axis_name="c", subcore_axis_name="s", num_cores=2, num_subcores=16
    )

    @functools.partial(
        pl.kernel,
        out_type=jax.ShapeDtypeStruct((8 * _BP,), jnp.float32),
        mesh=mesh,
        scratch_types=[
            pltpu.VMEM((_BT,), jnp.float32),
            pltpu.VMEM((_BT,), jnp.float32),
            pltpu.VMEM((_BT,), jnp.float32),
            pltpu.VMEM((_BT,), jnp.float32),
            pltpu.VMEM((_NG * 4, 128), jnp.int32),
            pltpu.VMEM((_NG * 4, 8, 128), jnp.float32),
            pltpu.VMEM((8, _BT), jnp.float32),
            pltpu.SemaphoreType.DMA,
            pltpu.SemaphoreType.DMA,
        ],
        compiler_params=pltpu.CompilerParams(
            needs_layout_passes=False, use_tc_tiling_on_sc=False
        ),
    )
    def enc(xs_h, ys_h, zs_h, ts_h,
            f0_h, f1_h, f2_h, f3_h, f4_h, f5_h, f6_h, f7_h, out_h,
            xs_v, ys_v, zs_v, ts_v, idx_v, rows_v, out_v, sem0, sem1):
        planes = (f0_h, f1_h, f2_h, f3_h, f4_h, f5_h, f6_h, f7_h)
        wid = lax.axis_index("s") * 2 + lax.axis_index("c")
        base = wid * _BT
        pltpu.sync_copy(xs_h.at[pl.ds(base, _BT)], xs_v)
        pltpu.sync_copy(ys_h.at[pl.ds(base, _BT)], ys_v)
        pltpu.sync_copy(zs_h.at[pl.ds(base, _BT)], zs_v)
        pltpu.sync_copy(ts_h.at[pl.ds(base, _BT)], ts_v)

        def load16(ref, b0):
            return ref[pl.ds(b0, 16)]

        def fracs(g):
            b0 = g * 16
            x = load16(xs_v, b0)
            y = load16(ys_v, b0)
            z = load16(zs_v, b0)
            out = []
            for l in range(_NL):
                r = _RES[l]
                inv = float(r - 1)
                px = x * inv
                py = y * inv
                pz = z * inv
                bx = jnp.clip(px.astype(jnp.int32), 0, r - 2)
                by = jnp.clip(py.astype(jnp.int32), 0, r - 2)
                bz = jnp.clip(pz.astype(jnp.int32), 0, r - 2)
                fx = px - bx.astype(jnp.float32)
                fy = py - by.astype(jnp.float32)
                fz = pz - bz.astype(jnp.float32)
                out.append((bx, by, bz, fx, fy, fz))
            return out

        def compute_indices(g):
            fr = fracs(g)
            for l in range(_NL):
                r = _RES[l]
                bx, by, bz, _, _, _ = fr[l]
                lin = bx * (r * r) + by * r + bz + _OFFS[l]
                row = g * 4 + l
                for c in range(8):
                    u, v, w = c >> 2, (c >> 1) & 1, c & 1
                    idx_v[row, pl.ds(c * 16, 16)] = lin + (u * (r * r) + v * r + w)

        def fire(g, sem):
            for l in range(_NL):
                row = g * 4 + l
                for e in range(8):
                    pltpu.async_copy(
                        planes[e].at[idx_v.at[row]],
                        rows_v.at[row, e],
                        sem,
                    )

        def drain(g, sem):
            for l in range(_NL):
                row = g * 4 + l
                for e in range(8):
                    pltpu.make_async_copy(
                        planes[e].at[idx_v.at[row]],
                        rows_v.at[row, e],
                        sem,
                    ).wait()

        def compute_out(g):
            b0 = g * 16
            t = load16(ts_v, b0)
            d = [t - float(ts_k) for ts_k in _TS]
            vb = [
                (d[1] * d[2] * d[3]) / float(_DENOM[0]),
                (d[0] * d[2] * d[3]) / float(_DENOM[1]),
                (d[0] * d[1] * d[3]) / float(_DENOM[2]),
                (d[0] * d[1] * d[2]) / float(_DENOM[3]),
            ]
            fr = fracs(g)
            for l in range(_NL):
                _, _, _, fx, fy, fz = fr[l]
                wx = [1.0 - fx, fx]
                wy = [1.0 - fy, fy]
                wz = [1.0 - fz, fz]
                row = g * 4 + l
                acc0 = jnp.zeros((16,), jnp.float32)
                acc1 = jnp.zeros((16,), jnp.float32)
                for c in range(8):
                    u, v, w = c >> 2, (c >> 1) & 1, c & 1
                    W = wx[u] * wy[v] * wz[w]
                    e = [rows_v[row, k, pl.ds(c * 16, 16)] for k in range(8)]
                    ft0 = vb[0] * e[0] + vb[1] * e[2] + vb[2] * e[4] + vb[3] * e[6]
                    ft1 = vb[0] * e[1] + vb[1] * e[3] + vb[2] * e[5] + vb[3] * e[7]
                    acc0 = acc0 + W * ft0
                    acc1 = acc1 + W * ft1
                out_v[2 * l, pl.ds(b0, 16)] = acc0
                out_v[2 * l + 1, pl.ds(b0, 16)] = acc1

        compute_indices(0)
        fire(0, sem0)
        compute_indices(1)
        fire(1, sem1)

        def body(i, carry):
            g = 2 * i
            drain(g, sem0)
            compute_indices(g + 2)
            fire(g + 2, sem0)
            compute_out(g)
            drain(g + 1, sem1)
            compute_indices(g + 3)
            fire(g + 3, sem1)
            compute_out(g + 1)
            return carry

        lax.fori_loop(0, (_NG - 2) // 2, body, 0)
        drain(_NG - 2, sem0)
        compute_out(_NG - 2)
        drain(_NG - 1, sem1)
        compute_out(_NG - 1)

        for rrow in range(8):
            pltpu.sync_copy(out_v.at[rrow], out_h.at[pl.ds(rrow * _BP + base, _BT)])

    return enc


_ENC_CACHE = []


@jax.jit
def kernel(positions, feats):
    if not _ENC_CACHE:
        _ENC_CACHE.append(_sc_encode_fn())
    pos = jnp.pad(positions, ((0, _BP - _B), (0, 0)))
    pos_t = pos.T
    planes = [feats[:, e] for e in range(8)]
    out_flat = _ENC_CACHE[0](pos_t[0], pos_t[1], pos_t[2], pos_t[3], *planes)
    out = out_flat.reshape(8, _BP).T
    return out[:_B]


# speedup vs baseline: 2.1511x; 1.0009x over previous
import functools

import jax
import jax.numpy as jnp
import numpy as np
from jax import lax
from jax.experimental import pallas as pl
from jax.experimental.pallas import tpu as pltpu
from jax.experimental.pallas import tpu_sc as plsc

_FEAT_DIM = 2
_TD = 4
_NL = 4
_RES = [16 * 2**l + 1 for l in range(_NL)]
_SIZES = [r * r * r for r in _RES]
_OFFS = [0]
for _s in _SIZES:
    _OFFS.append(_OFFS[-1] + _s)
_B = 10000
_NW = 32
_BT = 320
_BP = _NW * _BT
_NG = _BT // 16

_TS = np.arange(_TD, dtype=np.float32) * np.float32(1.0 / (_TD - 1))
_T1000 = np.array([_TS[1], _TS[0], _TS[0], _TS[0]], np.float32)
_T2211 = np.array([_TS[2], _TS[2], _TS[1], _TS[1]], np.float32)
_T3332 = np.array([_TS[3], _TS[3], _TS[3], _TS[2]], np.float32)
_DENOM = (_TS - _T1000) * (_TS - _T2211) * (_TS - _T3332)


def _sc_encode_fn():
    mesh = plsc.VectorSubcoreMesh(
        core_axis_name="c", subcore_axis_name="s", num_cores=2, num_subcores=16
    )

    @functools.partial(
        pl.kernel,
        out_type=jax.ShapeDtypeStruct((8 * _BP,), jnp.float32),
        mesh=mesh,
        scratch_types=[
            pltpu.VMEM((_BT,), jnp.float32),
            pltpu.VMEM((_BT,), jnp.float32),
            pltpu.VMEM((_BT,), jnp.float32),
            pltpu.VMEM((_BT,), jnp.float32),
            pltpu.VMEM((_NG * 512,), jnp.int32),
            pltpu.VMEM((_NG, 8, 512), jnp.float32),
            pltpu.VMEM((8, _BT), jnp.float32),
            pltpu.SemaphoreType.DMA,
            pltpu.SemaphoreType.DMA,
        ],
        compiler_params=pltpu.CompilerParams(
            needs_layout_passes=False, use_tc_tiling_on_sc=False
        ),
    )
    def enc(xs_h, ys_h, zs_h, ts_h,
            f0_h, f1_h, f2_h, f3_h, f4_h, f5_h, f6_h, f7_h, out_h,
            xs_v, ys_v, zs_v, ts_v, idx_v, rows_v, out_v, sem0, sem1):
        planes = (f0_h, f1_h, f2_h, f3_h, f4_h, f5_h, f6_h, f7_h)
        wid = lax.axis_index("s") * 2 + lax.axis_index("c")
        base = wid * _BT
        pltpu.sync_copy(xs_h.at[pl.ds(base, _BT)], xs_v)
        pltpu.sync_copy(ys_h.at[pl.ds(base, _BT)], ys_v)
        pltpu.sync_copy(zs_h.at[pl.ds(base, _BT)], zs_v)
        pltpu.sync_copy(ts_h.at[pl.ds(base, _BT)], ts_v)

        def load16(ref, b0):
            return ref[pl.ds(b0, 16)]

        def fracs(g):
            b0 = g * 16
            x = load16(xs_v, b0)
            y = load16(ys_v, b0)
            z = load16(zs_v, b0)
            out = []
            for l in range(_NL):
                r = _RES[l]
                inv = float(r - 1)
                px = x * inv
                py = y * inv
                pz = z * inv
                bx = jnp.clip(px.astype(jnp.int32), 0, r - 2)
                by = jnp.clip(py.astype(jnp.int32), 0, r - 2)
                bz = jnp.clip(pz.astype(jnp.int32), 0, r - 2)
                fx = px - bx.astype(jnp.float32)
                fy = py - by.astype(jnp.float32)
                fz = pz - bz.astype(jnp.float32)
                out.append((bx, by, bz, fx, fy, fz))
            return out

        def compute_indices(g):
            fr = fracs(g)
            for l in range(_NL):
                r = _RES[l]
                bx, by, bz, _, _, _ = fr[l]
                lin = bx * (r * r) + by * r + bz + _OFFS[l]
                gbase = g * 512 + l * 128
                for c in range(8):
                    u, v, w = c >> 2, (c >> 1) & 1, c & 1
                    idx_v[pl.ds(gbase + c * 16, 16)] = lin + (u * (r * r) + v * r + w)

        def fire(g, sem):
            for e in range(8):
                pltpu.async_copy(
                    planes[e].at[idx_v.at[pl.ds(g * 512, 512)]],
                    rows_v.at[g, e],
                    sem,
                )

        def drain(g, sem):
            for e in range(8):
                pltpu.make_async_copy(
                    planes[e].at[idx_v.at[pl.ds(g * 512, 512)]],
                    rows_v.at[g, e],
                    sem,
                ).wait()

        def compute_out(g):
            b0 = g * 16
            t = load16(ts_v, b0)
            d = [t - float(ts_k) for ts_k in _TS]
            vb = [
                (d[1] * d[2] * d[3]) / float(_DENOM[0]),
                (d[0] * d[2] * d[3]) / float(_DENOM[1]),
                (d[0] * d[1] * d[3]) / float(_DENOM[2]),
                (d[0] * d[1] * d[2]) / float(_DENOM[3]),
            ]
            fr = fracs(g)
            for l in range(_NL):
                _, _, _, fx, fy, fz = fr[l]
                wx = [1.0 - fx, fx]
                wy = [1.0 - fy, fy]
                wz = [1.0 - fz, fz]
                acc0 = jnp.zeros((16,), jnp.float32)
                acc1 = jnp.zeros((16,), jnp.float32)
                for c in range(8):
                    u, v, w = c >> 2, (c >> 1) & 1, c & 1
                    W = wx[u] * wy[v] * wz[w]
                    e = [
                        rows_v[g, n, pl.ds(l * 128 + c * 16, 16)]
                        for n in range(8)
                    ]
                    ft0 = vb[0] * e[0] + vb[1] * e[2] + vb[2] * e[4] + vb[3] * e[6]
                    ft1 = vb[0] * e[1] + vb[1] * e[3] + vb[2] * e[5] + vb[3] * e[7]
                    acc0 = acc0 + W * ft0
                    acc1 = acc1 + W * ft1
                out_v[2 * l, pl.ds(b0, 16)] = acc0
                out_v[2 * l + 1, pl.ds(b0, 16)] = acc1

        compute_indices(0)
        fire(0, sem0)
        compute_indices(1)
        fire(1, sem1)

        def body(i, carry):
            g = 2 * i
            drain(g, sem0)
            compute_indices(g + 2)
            fire(g + 2, sem0)
            compute_out(g)
            drain(g + 1, sem1)
            compute_indices(g + 3)
            fire(g + 3, sem1)
            compute_out(g + 1)
            return carry

        lax.fori_loop(0, (_NG - 2) // 2, body, 0)
        drain(_NG - 2, sem0)
        compute_out(_NG - 2)
        drain(_NG - 1, sem1)
        compute_out(_NG - 1)

        for rrow in range(8):
            pltpu.sync_copy(out_v.at[rrow], out_h.at[pl.ds(rrow * _BP + base, _BT)])

    return enc


_ENC_CACHE = []


@jax.jit
def kernel(positions, feats):
    if not _ENC_CACHE:
        _ENC_CACHE.append(_sc_encode_fn())
    pos = jnp.pad(positions, ((0, _BP - _B), (0, 0)))
    pos_t = pos.T
    planes = [feats[:, e] for e in range(8)]
    out_flat = _ENC_CACHE[0](pos_t[0], pos_t[1], pos_t[2], pos_t[3], *planes)
    out = out_flat.reshape(8, _BP).T
    return out[:_B]


# speedup vs baseline: 2.2863x; 1.0628x over previous
import functools

import jax
import jax.numpy as jnp
import numpy as np
from jax import lax
from jax.experimental import pallas as pl
from jax.experimental.pallas import tpu as pltpu
from jax.experimental.pallas import tpu_sc as plsc

_FEAT_DIM = 2
_TD = 4
_NL = 4
_RES = [16 * 2**l + 1 for l in range(_NL)]
_SIZES = [r * r * r for r in _RES]
_OFFS = [0]
for _s in _SIZES:
    _OFFS.append(_OFFS[-1] + _s)
_B = 10000
_NW = 32
_BT = 320
_BP = _NW * _BT
_NG = _BT // 16

_TS = np.arange(_TD, dtype=np.float32) * np.float32(1.0 / (_TD - 1))
_T1000 = np.array([_TS[1], _TS[0], _TS[0], _TS[0]], np.float32)
_T2211 = np.array([_TS[2], _TS[2], _TS[1], _TS[1]], np.float32)
_T3332 = np.array([_TS[3], _TS[3], _TS[3], _TS[2]], np.float32)
_DENOM = (_TS - _T1000) * (_TS - _T2211) * (_TS - _T3332)


def _sc_encode_fn(k0):
    mesh = plsc.VectorSubcoreMesh(
        core_axis_name="c", subcore_axis_name="s", num_cores=2, num_subcores=16
    )

    @functools.partial(
        pl.kernel,
        out_type=jax.ShapeDtypeStruct((8 * _BP,), jnp.float32),
        mesh=mesh,
        scratch_types=[
            pltpu.VMEM((_BT,), jnp.float32),
            pltpu.VMEM((_BT,), jnp.float32),
            pltpu.VMEM((_BT,), jnp.float32),
            pltpu.VMEM((_BT,), jnp.float32),
            pltpu.VMEM((_NG * 512,), jnp.int32),
            pltpu.VMEM((_NG, 4, 512), jnp.float32),
            pltpu.VMEM((8, _BT), jnp.float32),
            pltpu.SemaphoreType.DMA,
            pltpu.SemaphoreType.DMA,
        ],
        compiler_params=pltpu.CompilerParams(
            needs_layout_passes=False, use_tc_tiling_on_sc=False
        ),
    )
    def enc(xs_h, ys_h, zs_h, ts_h,
            f0_h, f1_h, f2_h, f3_h, out_h,
            xs_v, ys_v, zs_v, ts_v, idx_v, rows_v, out_v, sem0, sem1):
        planes = (f0_h, f1_h, f2_h, f3_h)
        wid = lax.axis_index("s") * 2 + lax.axis_index("c")
        base = wid * _BT
        pltpu.sync_copy(xs_h.at[pl.ds(base, _BT)], xs_v)
        pltpu.sync_copy(ys_h.at[pl.ds(base, _BT)], ys_v)
        pltpu.sync_copy(zs_h.at[pl.ds(base, _BT)], zs_v)
        pltpu.sync_copy(ts_h.at[pl.ds(base, _BT)], ts_v)

        def load16(ref, b0):
            return ref[pl.ds(b0, 16)]

        def fracs(g):
            b0 = g * 16
            x = load16(xs_v, b0)
            y = load16(ys_v, b0)
            z = load16(zs_v, b0)
            out = []
            for l in range(_NL):
                r = _RES[l]
                inv = float(r - 1)
                px = x * inv
                py = y * inv
                pz = z * inv
                bx = jnp.clip(px.astype(jnp.int32), 0, r - 2)
                by = jnp.clip(py.astype(jnp.int32), 0, r - 2)
                bz = jnp.clip(pz.astype(jnp.int32), 0, r - 2)
                fx = px - bx.astype(jnp.float32)
                fy = py - by.astype(jnp.float32)
                fz = pz - bz.astype(jnp.float32)
                out.append((bx, by, bz, fx, fy, fz))
            return out

        def compute_indices(g):
            fr = fracs(g)
            for l in range(_NL):
                r = _RES[l]
                bx, by, bz, _, _, _ = fr[l]
                lin = bx * (r * r) + by * r + bz + _OFFS[l]
                gbase = g * 512 + l * 128
                for c in range(8):
                    u, v, w = c >> 2, (c >> 1) & 1, c & 1
                    idx_v[pl.ds(gbase + c * 16, 16)] = lin + (u * (r * r) + v * r + w)

        def fire(g, sem):
            for e in range(4):
                pltpu.async_copy(
                    planes[e].at[idx_v.at[pl.ds(g * 512, 512)]],
                    rows_v.at[g, e],
                    sem,
                )

        def drain(g, sem):
            for e in range(4):
                pltpu.make_async_copy(
                    planes[e].at[idx_v.at[pl.ds(g * 512, 512)]],
                    rows_v.at[g, e],
                    sem,
                ).wait()

        def compute_out(g):
            b0 = g * 16
            t = load16(ts_v, b0)
            d = [t - float(ts_k) for ts_k in _TS]
            nums = [
                d[1] * d[2] * d[3],
                d[0] * d[2] * d[3],
                d[0] * d[1] * d[3],
                d[0] * d[1] * d[2],
            ]
            vb = [nums[k] / float(_DENOM[k]) for k in (k0, k0 + 1)]
            fr = fracs(g)
            for l in range(_NL):
                _, _, _, fx, fy, fz = fr[l]
                wx = [1.0 - fx, fx]
                wy = [1.0 - fy, fy]
                wz = [1.0 - fz, fz]
                acc0 = jnp.zeros((16,), jnp.float32)
                acc1 = jnp.zeros((16,), jnp.float32)
                for c in range(8):
                    u, v, w = c >> 2, (c >> 1) & 1, c & 1
                    W = wx[u] * wy[v] * wz[w]
                    e = [
                        rows_v[g, n, pl.ds(l * 128 + c * 16, 16)]
                        for n in range(4)
                    ]
                    ft0 = vb[0] * e[0] + vb[1] * e[2]
                    ft1 = vb[0] * e[1] + vb[1] * e[3]
                    acc0 = acc0 + W * ft0
                    acc1 = acc1 + W * ft1
                out_v[2 * l, pl.ds(b0, 16)] = acc0
                out_v[2 * l + 1, pl.ds(b0, 16)] = acc1

        compute_indices(0)
        fire(0, sem0)
        compute_indices(1)
        fire(1, sem1)

        def body(i, carry):
            g = 2 * i
            drain(g, sem0)
            compute_indices(g + 2)
            fire(g + 2, sem0)
            compute_out(g)
            drain(g + 1, sem1)
            compute_indices(g + 3)
            fire(g + 3, sem1)
            compute_out(g + 1)
            return carry

        lax.fori_loop(0, (_NG - 2) // 2, body, 0)
        drain(_NG - 2, sem0)
        compute_out(_NG - 2)
        drain(_NG - 1, sem1)
        compute_out(_NG - 1)

        for rrow in range(8):
            pltpu.sync_copy(out_v.at[rrow], out_h.at[pl.ds(rrow * _BP + base, _BT)])

    return enc


_ENC_CACHE = {}


@jax.jit
def kernel(positions, feats):
    if not _ENC_CACHE:
        _ENC_CACHE[0] = _sc_encode_fn(0)
        _ENC_CACHE[1] = _sc_encode_fn(2)
    pos = jnp.pad(positions, ((0, _BP - _B), (0, 0)))
    pos_t = pos.T
    planes = [feats[:, e] for e in range(8)]
    xs, ys, zs, ts = pos_t[0], pos_t[1], pos_t[2], pos_t[3]
    out_a = _ENC_CACHE[0](xs, ys, zs, ts, *planes[:4])
    out_b = _ENC_CACHE[1](xs, ys, zs, ts, *planes[4:])
    out = (out_a + out_b).reshape(8, _BP).T
    return out[:_B]


# speedup vs baseline: 4.7019x; 2.0565x over previous
import functools

import jax
import jax.numpy as jnp
import numpy as np
from jax import lax
from jax.experimental import pallas as pl
from jax.experimental.pallas import tpu as pltpu
from jax.experimental.pallas import tpu_sc as plsc

_FEAT_DIM = 2
_TD = 4
_NL = 4
_RES = [16 * 2**l + 1 for l in range(_NL)]
_SIZES = [r * r * r for r in _RES]
_OFFS = [0]
for _s in _SIZES:
    _OFFS.append(_OFFS[-1] + _s)
_B = 10000
_NW = 32
_BT = 320
_BP = _NW * _BT
_NG = _BT // 16

_TS = np.arange(_TD, dtype=np.float32) * np.float32(1.0 / (_TD - 1))
_T1000 = np.array([_TS[1], _TS[0], _TS[0], _TS[0]], np.float32)
_T2211 = np.array([_TS[2], _TS[2], _TS[1], _TS[1]], np.float32)
_T3332 = np.array([_TS[3], _TS[3], _TS[3], _TS[2]], np.float32)
_DENOM = (_TS - _T1000) * (_TS - _T2211) * (_TS - _T3332)

_N = _OFFS[-1]
_PCHUNK = 4096
_NFULL = 600
_TAIL0 = _NFULL * _PCHUNK
_TAILC = 2
_N8 = _TAIL0 + _TAILC * _PCHUNK
_REMW = _NW - 1


def _sc_prep_fn():
    mesh = plsc.VectorSubcoreMesh(
        core_axis_name="c", subcore_axis_name="s", num_cores=2, num_subcores=16
    )

    @functools.partial(
        pl.kernel,
        out_type=jax.ShapeDtypeStruct((8 * _N8,), jnp.float32),
        mesh=mesh,
        scratch_types=[
            pltpu.VMEM((2, 8, _PCHUNK), jnp.float32),
            pltpu.SemaphoreType.DMA,
            pltpu.SemaphoreType.DMA,
            pltpu.SemaphoreType.DMA,
            pltpu.SemaphoreType.DMA,
        ],
        compiler_params=pltpu.CompilerParams(
            needs_layout_passes=False, use_tc_tiling_on_sc=True
        ),
    )
    def prep(ft_h, tail_h, out_h, buf_v, si0, si1, so0, so1):
        wid = lax.axis_index("s") * 2 + lax.axis_index("c")
        n_w = (_NFULL - 1 - wid) // _NW + 1
        sin = (si0, si1)
        sout = (so0, so1)

        def c0_of(i):
            return (wid + i * _NW) * _PCHUNK

        def in_copy(i, par):
            return pltpu.make_async_copy(
                ft_h.at[:, pl.ds(c0_of(i), _PCHUNK)], buf_v.at[par], sin[par]
            )

        def out_copy(i, par, e):
            return pltpu.make_async_copy(
                buf_v.at[par, e],
                out_h.at[pl.ds(e * _N8 + c0_of(i), _PCHUNK)],
                sout[par],
            )

        def step(i, par):
            @pl.when(i < n_w)
            def _():
                @pl.when(i >= 2)
                def _():
                    for e in range(8):
                        out_copy(i - 2, par, e).wait()

                in_copy(i, par).start()
                in_copy(i, par).wait()
                for e in range(8):
                    out_copy(i, par, e).start()

        def body(j, carry):
            step(2 * j, 0)
            step(2 * j + 1, 1)
            return carry

        lax.fori_loop(0, _NFULL // _NW // 2 + 1, body, 0)

        _w19 = (_NFULL - 1) % _NW

        @pl.when(wid <= _w19)
        def _():
            for e in range(8):
                out_copy(17, 1, e).wait()
            for e in range(8):
                out_copy(18, 0, e).wait()

        @pl.when(wid > _w19)
        def _():
            for e in range(8):
                out_copy(16, 0, e).wait()
            for e in range(8):
                out_copy(17, 1, e).wait()

        @pl.when(wid == _REMW)
        def _():
            for q in range(_TAILC):
                pltpu.sync_copy(tail_h.at[:, pl.ds(q * _PCHUNK, _PCHUNK)], buf_v.at[0])
                for e in range(8):
                    pltpu.sync_copy(
                        buf_v.at[0, e],
                        out_h.at[pl.ds(e * _N8 + _TAIL0 + q * _PCHUNK, _PCHUNK)],
                    )

    return prep


def _sc_encode_fn():
    mesh = plsc.VectorSubcoreMesh(
        core_axis_name="c", subcore_axis_name="s", num_cores=2, num_subcores=16
    )

    @functools.partial(
        pl.kernel,
        out_type=jax.ShapeDtypeStruct((8 * _BP,), jnp.float32),
        mesh=mesh,
        scratch_types=[
            pltpu.VMEM((_BT,), jnp.float32),
            pltpu.VMEM((_BT,), jnp.float32),
            pltpu.VMEM((_BT,), jnp.float32),
            pltpu.VMEM((_BT,), jnp.float32),
            pltpu.VMEM((_NG * 512,), jnp.int32),
            pltpu.VMEM((_NG, 8, 512), jnp.float32),
            pltpu.VMEM((8, _BT), jnp.float32),
            pltpu.SemaphoreType.DMA,
            pltpu.SemaphoreType.DMA,
        ],
        compiler_params=pltpu.CompilerParams(
            needs_layout_passes=False, use_tc_tiling_on_sc=False
        ),
    )
    def enc(xs_h, ys_h, zs_h, ts_h, pf_h, out_h,
            xs_v, ys_v, zs_v, ts_v, idx_v, rows_v, out_v, sem0, sem1):
        planes = tuple(pf_h.at[pl.ds(e * _N8, _N8)] for e in range(8))
        wid = lax.axis_index("s") * 2 + lax.axis_index("c")
        base = wid * _BT
        pltpu.sync_copy(xs_h.at[pl.ds(base, _BT)], xs_v)
        pltpu.sync_copy(ys_h.at[pl.ds(base, _BT)], ys_v)
        pltpu.sync_copy(zs_h.at[pl.ds(base, _BT)], zs_v)
        pltpu.sync_copy(ts_h.at[pl.ds(base, _BT)], ts_v)

        def load16(ref, b0):
            return ref[pl.ds(b0, 16)]

        def fracs(g):
            b0 = g * 16
            x = load16(xs_v, b0)
            y = load16(ys_v, b0)
            z = load16(zs_v, b0)
            out = []
            for l in range(_NL):
                r = _RES[l]
                inv = float(r - 1)
                px = x * inv
                py = y * inv
                pz = z * inv
                bx = jnp.clip(px.astype(jnp.int32), 0, r - 2)
                by = jnp.clip(py.astype(jnp.int32), 0, r - 2)
                bz = jnp.clip(pz.astype(jnp.int32), 0, r - 2)
                fx = px - bx.astype(jnp.float32)
                fy = py - by.astype(jnp.float32)
                fz = pz - bz.astype(jnp.float32)
                out.append((bx, by, bz, fx, fy, fz))
            return out

        def compute_indices(g):
            fr = fracs(g)
            for l in range(_NL):
                r = _RES[l]
                bx, by, bz, _, _, _ = fr[l]
                lin = bx * (r * r) + by * r + bz + _OFFS[l]
                gbase = g * 512 + l * 128
                for c in range(8):
                    u, v, w = c >> 2, (c >> 1) & 1, c & 1
                    idx_v[pl.ds(gbase + c * 16, 16)] = lin + (u * (r * r) + v * r + w)

        def fire(g, sem):
            for e in range(8):
                pltpu.async_copy(
                    planes[e].at[idx_v.at[pl.ds(g * 512, 512)]],
                    rows_v.at[g, e],
                    sem,
                )

        def drain(g, sem):
            for e in range(8):
                pltpu.make_async_copy(
                    planes[e].at[idx_v.at[pl.ds(g * 512, 512)]],
                    rows_v.at[g, e],
                    sem,
                ).wait()

        def compute_out(g):
            b0 = g * 16
            t = load16(ts_v, b0)
            d = [t - float(ts_k) for ts_k in _TS]
            vb = [
                (d[1] * d[2] * d[3]) / float(_DENOM[0]),
                (d[0] * d[2] * d[3]) / float(_DENOM[1]),
                (d[0] * d[1] * d[3]) / float(_DENOM[2]),
                (d[0] * d[1] * d[2]) / float(_DENOM[3]),
            ]
            fr = fracs(g)
            for l in range(_NL):
                _, _, _, fx, fy, fz = fr[l]
                wx = [1.0 - fx, fx]
                wy = [1.0 - fy, fy]
                wz = [1.0 - fz, fz]
                acc0 = jnp.zeros((16,), jnp.float32)
                acc1 = jnp.zeros((16,), jnp.float32)
                for c in range(8):
                    u, v, w = c >> 2, (c >> 1) & 1, c & 1
                    W = wx[u] * wy[v] * wz[w]
                    e = [
                        rows_v[g, n, pl.ds(l * 128 + c * 16, 16)]
                        for n in range(8)
                    ]
                    ft0 = vb[0] * e[0] + vb[1] * e[2] + vb[2] * e[4] + vb[3] * e[6]
                    ft1 = vb[0] * e[1] + vb[1] * e[3] + vb[2] * e[5] + vb[3] * e[7]
                    acc0 = acc0 + W * ft0
                    acc1 = acc1 + W * ft1
                out_v[2 * l, pl.ds(b0, 16)] = acc0
                out_v[2 * l + 1, pl.ds(b0, 16)] = acc1

        compute_indices(0)
        fire(0, sem0)
        compute_indices(1)
        fire(1, sem1)

        def body(i, carry):
            g = 2 * i
            drain(g, sem0)
            compute_indices(g + 2)
            fire(g + 2, sem0)
            compute_out(g)
            drain(g + 1, sem1)
            compute_indices(g + 3)
            fire(g + 3, sem1)
            compute_out(g + 1)
            return carry

        lax.fori_loop(0, (_NG - 2) // 2, body, 0)
        drain(_NG - 2, sem0)
        compute_out(_NG - 2)
        drain(_NG - 1, sem1)
        compute_out(_NG - 1)

        for rrow in range(8):
            pltpu.sync_copy(out_v.at[rrow], out_h.at[pl.ds(rrow * _BP + base, _BT)])

    return enc


_ENC_CACHE = {}


@jax.jit
def kernel(positions, feats):
    if not _ENC_CACHE:
        _ENC_CACHE["prep"] = _sc_prep_fn()
        _ENC_CACHE["enc"] = _sc_encode_fn()
    pos = jnp.pad(positions, ((0, _BP - _B), (0, 0)))
    pos_t = pos.T
    tail = jnp.pad(
        feats[_TAIL0:], ((0, _TAILC * _PCHUNK - (_N - _TAIL0)), (0, 0))
    ).T
    planes_flat = _ENC_CACHE["prep"](feats.T, tail)
    out_flat = _ENC_CACHE["enc"](
        pos_t[0], pos_t[1], pos_t[2], pos_t[3], planes_flat
    )
    out = out_flat.reshape(8, _BP).T
    return out[:_B]
